# baseline (speedup 1.0000x reference)
import functools

import jax
import jax.numpy as jnp
from jax import lax
from jax.experimental import pallas as pl
from jax.experimental.pallas import tpu as pltpu
from jax.experimental.pallas import tpu_sc as plsc

_LANES = 16
_NC = 2
_NS = 16
_CP = 48
_CH = 80
_HW = _CP // 2


def _transpose_tc(w48):
    cp, n = w48.shape
    hw = cp // 2

    def body(in_ref, out_ref):
        x = in_ref[...]
        out_ref[...] = jnp.stack([x[:hw].T, x[hw:].T])

    return pl.pallas_call(
        body,
        out_shape=jax.ShapeDtypeStruct((2, n, hw), w48.dtype),
    )(w48)


def _sc_link(ei, wt, b, co):
    cpt = ei.shape[2]
    n = wt.shape[1]
    rpt = (n // _NS) // 8 * 8
    rem = n - _NS * rpt
    zrows = 2 * n // _NS // 5

    mesh = plsc.VectorSubcoreMesh(core_axis_name="c", subcore_axis_name="s")

    @functools.partial(
        pl.kernel,
        mesh=mesh,
        out_type=jax.ShapeDtypeStruct((n, co), jnp.float32),
        scratch_types=[
            pltpu.VMEM_SHARED((2 * n, _HW), jnp.float32),
            pltpu.VMEM_SHARED((_NS, _LANES), jnp.int32),
            pltpu.VMEM((cpt, _CH), jnp.int32),
            pltpu.VMEM((cpt, _CH), jnp.int32),
            [pltpu.VMEM((_CH, _HW), jnp.float32)] * 8,
            pltpu.VMEM((zrows, _HW), jnp.float32),
            pltpu.VMEM((_LANES,), jnp.int32),
            pltpu.VMEM((_NS, _LANES), jnp.int32),
            pltpu.VMEM((_HW,), jnp.float32),
            pltpu.VMEM((rpt, _HW), jnp.float32),
            pltpu.VMEM((rpt, _HW), jnp.float32),
            [pltpu.SemaphoreType.DMA] * 8,
            [pltpu.SemaphoreType.DMA] * 8,
        ],
        compiler_params=pltpu.CompilerParams(use_tc_tiling_on_sc=False),
    )
    def k(ei_ref, wt_ref, b_ref, out_ref, acc_s, min_s, colb, rowb, msgs,
          zbuf, minv, mb, bb, av, ob, gsems, ssems):
        c = lax.axis_index("c")
        s = lax.axis_index("s")
        coff = c * _HW

        nbuf = 8
        k_ = 4

        def gather(j, bi):
            pltpu.async_copy(wt_ref.at[c].at[colb.at[j]], msgs[bi], gsems[bi])

        def swait(bi):
            pltpu.make_async_copy(
                msgs[bi], acc_s.at[rowb.at[0]], ssems[bi]).wait()

        pltpu.sync_copy(ei_ref.at[1, s], colb)
        pltpu.sync_copy(ei_ref.at[0, s], rowb)
        for j in range(k_):
            gather(j, j)

        zero = jnp.zeros((_LANES,), jnp.float32)

        def zrow(r, carry):
            zbuf[r, pl.ds(0, _LANES)] = zero
            zbuf[r, pl.ds(_HW - _LANES, _LANES)] = zero
            return carry

        lax.fori_loop(0, zrows, zrow, 0)
        for t in range(5):
            pltpu.sync_copy(
                zbuf, acc_s.at[pl.ds((5 * s + t) * zrows, zrows), :])

        def mrow(j, mm):
            for u in range(_CH // _LANES):
                mm = jnp.minimum(mm, rowb[j, pl.ds(u * _LANES, _LANES)])
            return mm

        mm = lax.fori_loop(
            0, cpt, mrow,
            jnp.full((_LANES,), jnp.iinfo(jnp.int32).max, jnp.int32))
        minv[...] = mm
        pltpu.sync_copy(minv, min_s.at[s])

        plsc.subcore_barrier()

        def step(i, bi):
            bn = (bi + k_) % nbuf
            pltpu.make_async_copy(
                wt_ref.at[c].at[colb.at[i]], msgs[bi], gsems[bi]).wait()
            pltpu.async_copy(msgs[bi], acc_s.at[rowb.at[i]], ssems[bi],
                             add=True)

            @pl.when(i >= nbuf - k_)
            def _settle():
                swait(bn)

            @pl.when(i + k_ < cpt)
            def _refill():
                gather(i + k_, bn)

        def group(g, carry):
            for bi in range(nbuf):
                step(g * nbuf + bi, bi)
            return carry

        lax.fori_loop(0, cpt // nbuf, group, 0)
        for i in range(cpt - cpt % nbuf, cpt):
            step(i, i % nbuf)
        for t in range(nbuf - k_):
            swait((cpt - (nbuf - k_) + t) % nbuf)

        plsc.subcore_barrier()

        pltpu.sync_copy(min_s, mb)

        def mfold(i, mm2):
            return jnp.minimum(mm2, mb[i, :])

        mm2 = lax.fori_loop(
            0, _NS, mfold,
            jnp.full((_LANES,), jnp.iinfo(jnp.int32).max, jnp.int32))
        m = mm2[0]
        for j in range(1, _LANES):
            m = jnp.minimum(m, mm2[j])

        pltpu.sync_copy(b_ref.at[pl.ds(coff, _HW)], bb.at[pl.ds(0, _HW)])
        bA = bb[pl.ds(0, _LANES)]
        bB = bb[pl.ds(_HW - _LANES, _LANES)]
        nvalid = n - m

        def epi(r0, nr):
            pltpu.sync_copy(acc_s.at[pl.ds(m + r0, nr), :],
                            av.at[pl.ds(0, nr), :])

            def erow(g, carry):
                valid = (r0 + g) < nvalid
                vA = av[g, pl.ds(0, _LANES)]
                vB = av[g, pl.ds(_HW - _LANES, _LANES)]
                vA = jnp.where(valid, vA, jnp.zeros_like(vA)) + bA
                vB = jnp.where(valid, vB, jnp.zeros_like(vB)) + bB
                ob[g, pl.ds(0, _LANES)] = vA
                ob[g, pl.ds(_HW - _LANES, _LANES)] = vB
                return carry

            lax.fori_loop(0, nr, erow, 0)

            @pl.when(c == 0)
            def _store0():
                pltpu.sync_copy(ob.at[pl.ds(0, nr), :],
                                out_ref.at[pl.ds(r0, nr), pl.ds(0, _HW)])

            @pl.when(c == 1)
            def _store1():
                pltpu.sync_copy(
                    ob.at[pl.ds(0, nr), pl.ds(0, co - _HW)],
                    out_ref.at[pl.ds(r0, nr), pl.ds(_HW, co - _HW)])

        epi(s * rpt, rpt)

        @pl.when(s == 0)
        def _epi_tail():
            epi(_NS * rpt, rem)

    b_pad = jnp.concatenate([b, jnp.zeros((_CP - b.shape[0],), b.dtype)])
    return k(ei, wt, b_pad)


def kernel(x, edge_index, W, b):
    del x
    c, n = W.shape
    e = edge_index.shape[1]

    w48 = jnp.concatenate(
        [W, jnp.zeros((_CP - c, n), W.dtype)], axis=0)
    ei = edge_index.reshape(2, _NS, e // (_NS * _CH), _CH)

    wt = _transpose_tc(w48)
    return _sc_link(ei, wt, b, c)
